# baseline (speedup 1.0000x reference)
import functools

import jax
import jax.numpy as jnp
from jax import lax
from jax.experimental import pallas as pl
from jax.experimental.pallas import tpu as pltpu
from jax.experimental.pallas import tpu_sc as plsc

N_NODES = 10000
N_EDGES = 320000
N_GRAPHS = 16
D = 128

NC = 2
NS = 16
NW = NC * NS
K = 128
C = 80
E_PAD = NW * C * K
N_PAD = 10240
TR = N_PAD // NW
STRIPE = N_PAD // NS
E2 = E_PAD + 32 * 128
MAGIC = 6554

_mesh = lambda: plsc.VectorSubcoreMesh(
    core_axis_name="c", subcore_axis_name="s", num_cores=NC, num_subcores=NS)


def _i16():
  return lax.iota(jnp.int32, 16)


def _lane_sum(v):
  t = v[0]
  for k in range(1, 16):
    t = t + v[k]
  return t


def _popcount(m):
  return _lane_sum(jnp.where(m, 1, 0))


def _dyn_lane(v_lo, v_hi, b):
  i16 = _i16()
  sel = (jnp.where(i16 == b, v_lo, 0) +
         jnp.where(i16 == (b - 16), v_hi, 0))
  return _lane_sum(sel)


def _sc_deg(dst3, w3):
  @functools.partial(
      pl.kernel,
      out_type=jax.ShapeDtypeStruct((NC, N_PAD), jnp.float32),
      mesh=_mesh(),
      scratch_types=[
          pltpu.VMEM((C, K), jnp.int32),
          pltpu.VMEM((C, K), jnp.float32),
          pltpu.VMEM((STRIPE,), jnp.float32),
          pltpu.VMEM_SHARED((N_PAD,), jnp.float32),
          pltpu.SemaphoreType.DMA,
      ],
  )
  def k(dst_hbm, w_hbm, out_hbm, idx_v, w_v, zero_v, acc_sh, sem):
    c = lax.axis_index("c")
    s = lax.axis_index("s")
    w = s * NC + c
    z16 = jnp.zeros((16,), jnp.float32)

    def zinit(i, _):
      zero_v[pl.ds(i * 16, 16)] = z16
      return 0

    lax.fori_loop(0, STRIPE // 16, zinit, 0)
    pltpu.sync_copy(zero_v, acc_sh.at[pl.ds(s * STRIPE, STRIPE)])
    plsc.subcore_barrier()

    pltpu.sync_copy(dst_hbm.at[w], idx_v)
    pltpu.sync_copy(w_hbm.at[w], w_v)

    def body(bi, _):
      for j in range(8):
        i = bi * 8 + j
        pltpu.async_copy(w_v.at[i], acc_sh.at[idx_v.at[i]], sem, add=True)
      for j in range(8):
        pltpu.make_async_copy(w_v.at[0], acc_sh.at[idx_v.at[0]], sem).wait()
      return 0

    lax.fori_loop(0, C // 8, body, 0)
    plsc.subcore_barrier()
    pltpu.sync_copy(acc_sh.at[pl.ds(s * STRIPE, STRIPE)],
                    out_hbm.at[c, pl.ds(s * STRIPE, STRIPE)])

  return k(dst3, w3)


def _sc_count(dst3):
  @functools.partial(
      pl.kernel,
      out_type=jax.ShapeDtypeStruct((NW, 32), jnp.int32),
      mesh=_mesh(),
      scratch_types=[
          pltpu.VMEM((C, K), jnp.int32),
          pltpu.VMEM((32,), jnp.int32),
      ],
  )
  def k(dst_hbm, out_hbm, dst_v, cnt_v):
    c = lax.axis_index("c")
    s = lax.axis_index("s")
    w = s * NC + c
    pltpu.sync_copy(dst_hbm.at[w], dst_v)
    i16 = _i16()
    zero = jnp.zeros((16,), jnp.int32)
    totals = []
    for p in range(4):
      def scan(i, accs):
        accs = list(accs)
        for j in range(8):
          dv = dst_v[i, pl.ds(j * 16, 16)]
          bv = (dv * MAGIC) >> 21
          for t in range(8):
            accs[t] = accs[t] + jnp.where(bv == (8 * p + t), 1, 0)
        return tuple(accs)

      accs = lax.fori_loop(0, C, scan, tuple([zero] * 8))
      totals.extend(_lane_sum(a) for a in accs)
    lo = jnp.zeros((16,), jnp.int32)
    hi = jnp.zeros((16,), jnp.int32)
    for b in range(16):
      lo = jnp.where(i16 == b, totals[b], lo)
      hi = jnp.where(i16 == b, totals[16 + b], hi)
    cnt_v[pl.ds(0, 16)] = lo
    cnt_v[pl.ds(16, 16)] = hi
    pltpu.sync_copy(cnt_v, out_hbm.at[w])

  return k(dst3)


def _sc_bucket(src3, dst3, w3, off2, tstart, tlen):
  @functools.partial(
      pl.kernel,
      out_type=(
          jax.ShapeDtypeStruct((E2 + 16,), jnp.int32),
          jax.ShapeDtypeStruct((E2 + 16,), jnp.int32),
          jax.ShapeDtypeStruct((E2 + 16,), jnp.float32),
      ),
      mesh=_mesh(),
      scratch_types=[
          pltpu.VMEM((C, K), jnp.int32),
          pltpu.VMEM((C, K), jnp.int32),
          pltpu.VMEM((C, K), jnp.float32),
          pltpu.VMEM((K,), jnp.int32),
          pltpu.VMEM((K,), jnp.int32),
          pltpu.VMEM((K,), jnp.int32),
          pltpu.VMEM((K,), jnp.float32),
          pltpu.VMEM((32,), jnp.int32),
          pltpu.VMEM((32,), jnp.int32),
          pltpu.VMEM((32,), jnp.int32),
          pltpu.SMEM((32,), jnp.int32),
          pltpu.SemaphoreType.DMA,
      ],
  )
  def k(src_hbm, dst_hbm, w_hbm, off_hbm, ts_hbm, tl_hbm,
        bsrc_hbm, bloc_hbm, bw_hbm,
        src_v, dst_v, w_v, locb, posb, zib, zfb, off_v, ts_v, tl_v,
        smem_pos, sem):
    c = lax.axis_index("c")
    s = lax.axis_index("s")
    w = s * NC + c
    pltpu.sync_copy(src_hbm.at[w], src_v)
    pltpu.sync_copy(dst_hbm.at[w], dst_v)
    pltpu.sync_copy(w_hbm.at[w], w_v)
    pltpu.sync_copy(off_hbm.at[w], off_v)
    pltpu.sync_copy(ts_hbm, ts_v)
    pltpu.sync_copy(tl_hbm, tl_v)
    off_lo = off_v[pl.ds(0, 16)]
    off_hi = off_v[pl.ds(16, 16)]
    i16 = _i16()
    z16i = jnp.zeros((16,), jnp.int32)
    z16f = jnp.zeros((16,), jnp.float32)
    for b in range(16):
      smem_pos[b] = off_lo[b]
      smem_pos[16 + b] = off_hi[b]
    for j in range(8):
      zib[pl.ds(j * 16, 16)] = z16i
      zfb[pl.ds(j * 16, 16)] = z16f

    def chunk(i, _):
      for j in range(8):
        dv = dst_v[i, pl.ds(j * 16, 16)]
        bv = (dv * MAGIC) >> 21
        locb[pl.ds(j * 16, 16)] = dv - bv * 320
        posv = z16i
        for l in range(16):
          bs = bv[l]
          p = smem_pos[bs]
          smem_pos[bs] = p + 1
          posv = jnp.where(i16 == l, p, posv)
        posb[pl.ds(j * 16, 16)] = posv
      a1 = pltpu.async_copy(src_v.at[i], bsrc_hbm.at[posb], sem)
      a2 = pltpu.async_copy(locb, bloc_hbm.at[posb], sem)
      a3 = pltpu.async_copy(w_v.at[i], bw_hbm.at[posb], sem)
      a1.wait()
      a2.wait()
      a3.wait()
      return 0

    lax.fori_loop(0, C, chunk, 0)

    tsw = _dyn_lane(ts_v[pl.ds(0, 16)], ts_v[pl.ds(16, 16)], w)
    tlw = _dyn_lane(tl_v[pl.ds(0, 16)], tl_v[pl.ds(16, 16)], w)
    for j in range(8):
      idx = 16 * j + i16
      posb[pl.ds(j * 16, 16)] = jnp.where(idx < tlw, tsw + idx, E2)
    a1 = pltpu.async_copy(zib, bsrc_hbm.at[posb], sem)
    a2 = pltpu.async_copy(zib, bloc_hbm.at[posb], sem)
    a3 = pltpu.async_copy(zfb, bw_hbm.at[posb], sem)
    a1.wait()
    a2.wait()
    a3.wait()

  return k(src3, dst3, w3, off2, tstart, tlen)


def _sc_spmm(y, bsrc, bloc, bw, bstart, bchunks):
  @functools.partial(
      pl.kernel,
      out_type=jax.ShapeDtypeStruct((NW, TR, D), jnp.float32),
      mesh=_mesh(),
      scratch_types=[
          pltpu.VMEM((TR, D), jnp.float32),
          pltpu.VMEM((K,), jnp.int32),
          pltpu.VMEM((K,), jnp.int32),
          pltpu.VMEM((K,), jnp.float32),
          pltpu.VMEM((K, D), jnp.float32),
          pltpu.VMEM((K, D), jnp.float32),
          pltpu.VMEM((32,), jnp.int32),
          pltpu.VMEM((32,), jnp.int32),
          pltpu.SemaphoreType.DMA,
          pltpu.SemaphoreType.DMA,
          pltpu.SemaphoreType.DMA,
      ],
  )
  def k(y_hbm, bsrc_hbm, bloc_hbm, bw_hbm, bs_hbm, bc_hbm, out_hbm,
        acc_v, sidx_v, loc_v, wq_v, rows0, rows1, bs_v, bc_v,
        semg0, semg1, sems):
    c = lax.axis_index("c")
    s = lax.axis_index("s")
    w = s * NC + c
    pltpu.sync_copy(bs_hbm, bs_v)
    pltpu.sync_copy(bc_hbm, bc_v)
    start = _dyn_lane(bs_v[pl.ds(0, 16)], bs_v[pl.ds(16, 16)], w)
    nq = _dyn_lane(bc_v[pl.ds(0, 16)], bc_v[pl.ds(16, 16)], w)
    z16 = jnp.zeros((16,), jnp.float32)

    def zinit(i, _):
      for j in range(D // 16):
        acc_v[i, pl.ds(j * 16, 16)] = z16
      return 0

    lax.fori_loop(0, TR, zinit, 0)

    def accumulate(buf):
      def srb(rb, _):
        nv16 = wq_v[pl.ds(rb * 16, 16)]
        lv16 = loc_v[pl.ds(rb * 16, 16)]
        for rr in range(16):
          r = rb * 16 + rr
          nv = nv16[rr]
          lr = lv16[rr]
          for j in range(D // 16):
            acc_v[lr, pl.ds(j * 16, 16)] = (
                acc_v[lr, pl.ds(j * 16, 16)] +
                buf[r, pl.ds(j * 16, 16)] * nv)
        return 0

      lax.fori_loop(0, K // 16, srb, 0)

    def chunk(q, prev):
      base = pl.multiple_of(start + q * K, K)
      pltpu.sync_copy(bsrc_hbm.at[pl.ds(base, K)], sidx_v)
      pltpu.sync_copy(bloc_hbm.at[pl.ds(base, K)], loc_v)
      pltpu.sync_copy(bw_hbm.at[pl.ds(base, K)], wq_v)
      pltpu.async_copy(y_hbm.at[sidx_v], rows0, semg0).wait()
      accumulate(rows0)
      return prev

    lax.fori_loop(0, nq, chunk, 0)
    pltpu.sync_copy(acc_v, out_hbm.at[w])

  return k(y, bsrc, bloc, bw, bstart, bchunks)


_RB = 400
_NB = N_NODES // _RB


def _tc_matmul(x, W, dv):
  def body(x_ref, w_ref, dv_ref, o_ref):
    o_ref[...] = jnp.dot(x_ref[...], w_ref[...],
                         preferred_element_type=jnp.float32) * dv_ref[...]

  return pl.pallas_call(
      body,
      grid=(_NB,),
      in_specs=[
          pl.BlockSpec((_RB, D), lambda i: (i, 0)),
          pl.BlockSpec((D, D), lambda i: (0, 0)),
          pl.BlockSpec((_RB, 1), lambda i: (i, 0)),
      ],
      out_specs=pl.BlockSpec((_RB, D), lambda i: (i, 0)),
      out_shape=jax.ShapeDtypeStruct((N_NODES, D), jnp.float32),
  )(x, W, dv)


def _tc_mid(p, y, dv, b, W2):
  def body(p_ref, y_ref, dv_ref, b_ref, w2_ref, o_ref):
    h = (p_ref[...] + y_ref[...]) * dv_ref[...] + b_ref[...]
    h = jnp.maximum(h, 0.0)
    o_ref[...] = jnp.dot(h, w2_ref[...],
                         preferred_element_type=jnp.float32) * dv_ref[...]

  return pl.pallas_call(
      body,
      grid=(_NB,),
      in_specs=[
          pl.BlockSpec((_RB, D), lambda i: (i, 0)),
          pl.BlockSpec((_RB, D), lambda i: (i, 0)),
          pl.BlockSpec((_RB, 1), lambda i: (i, 0)),
          pl.BlockSpec((1, D), lambda i: (0, 0)),
          pl.BlockSpec((D, D), lambda i: (0, 0)),
      ],
      out_specs=pl.BlockSpec((_RB, D), lambda i: (i, 0)),
      out_shape=jax.ShapeDtypeStruct((N_NODES, D), jnp.float32),
  )(p, y, dv, b, W2)


def _tc_final(p, y, dv, b, bid, speed, Ws, bs, Wl1g, Wl1v, bl1, Wl2, bl2):
  def body(p_ref, y_ref, dv_ref, b_ref, bid_ref, speed_ref, ws_ref,
           bs_ref, wl1g_ref, wl1v_ref, bl1_ref, wl2_ref, bl2_ref, o_ref,
           acc_ref):
    i = pl.program_id(0)

    @pl.when(i == 0)
    def _():
      acc_ref[...] = jnp.full((N_GRAPHS, D), -jnp.inf, jnp.float32)

    h = (p_ref[...] + y_ref[...]) * dv_ref[...] + b_ref[...]
    h = jnp.maximum(h, 0.0)
    bid = bid_ref[...]
    for g in range(N_GRAPHS):
      mg = jnp.max(jnp.where(bid == g, h, -jnp.inf), axis=0, keepdims=True)
      acc_ref[pl.ds(g, 1), :] = jnp.maximum(acc_ref[pl.ds(g, 1), :], mg)

    @pl.when(i == _NB - 1)
    def _():
      gmax = acc_ref[...]
      v = speed_ref[...] * ws_ref[...] + bs_ref[...]
      hh = jnp.dot(gmax, wl1g_ref[...], preferred_element_type=jnp.float32)
      hh = hh + jnp.dot(v, wl1v_ref[...], preferred_element_type=jnp.float32)
      hh = jnp.maximum(hh + bl1_ref[...], 0.0)
      o_ref[...] = jnp.dot(hh, wl2_ref[...],
                           preferred_element_type=jnp.float32) + bl2_ref[...]

  return pl.pallas_call(
      body,
      grid=(_NB,),
      in_specs=[
          pl.BlockSpec((_RB, D), lambda i: (i, 0)),
          pl.BlockSpec((_RB, D), lambda i: (i, 0)),
          pl.BlockSpec((_RB, 1), lambda i: (i, 0)),
          pl.BlockSpec((1, D), lambda i: (0, 0)),
          pl.BlockSpec((_RB, 1), lambda i: (i, 0)),
          pl.BlockSpec((N_GRAPHS, 1), lambda i: (0, 0)),
          pl.BlockSpec((1, 4), lambda i: (0, 0)),
          pl.BlockSpec((1, 4), lambda i: (0, 0)),
          pl.BlockSpec((D, N_GRAPHS), lambda i: (0, 0)),
          pl.BlockSpec((4, N_GRAPHS), lambda i: (0, 0)),
          pl.BlockSpec((1, N_GRAPHS), lambda i: (0, 0)),
          pl.BlockSpec((N_GRAPHS, N_GRAPHS), lambda i: (0, 0)),
          pl.BlockSpec((1, N_GRAPHS), lambda i: (0, 0)),
      ],
      out_specs=pl.BlockSpec((N_GRAPHS, N_GRAPHS), lambda i: (0, 0)),
      out_shape=jax.ShapeDtypeStruct((N_GRAPHS, N_GRAPHS), jnp.float32),
      scratch_shapes=[pltpu.VMEM((N_GRAPHS, D), jnp.float32)],
  )(p, y, dv, b, bid, speed, Ws, bs, Wl1g, Wl1v, bl1, Wl2, bl2)


def kernel(x, edge_index, edge_weight, batch_ids, speed,
           W1, b1, W2, b2, Ws, bs, Wl1, bl1, Wl2, bl2):
  src = edge_index[0].astype(jnp.int32)
  dst = edge_index[1].astype(jnp.int32)
  ew = edge_weight.astype(jnp.float32)
  pad = E_PAD - N_EDGES
  pad_dst = (jnp.arange(pad, dtype=jnp.int32) * 13) % N_NODES
  src3 = jnp.pad(src, (0, pad)).reshape(NW, C, K)
  dst3 = jnp.concatenate([dst, pad_dst]).reshape(NW, C, K)
  w3 = jnp.pad(ew, (0, pad)).reshape(NW, C, K)
  bid = batch_ids.astype(jnp.int32).reshape(N_NODES, 1)

  cnt = _sc_count(dst3)
  piece = jnp.cumsum(cnt, axis=0) - cnt
  used = jnp.sum(cnt, axis=0)
  blen = ((used + 127) // 128) * 128
  bstart = (jnp.cumsum(blen) - blen).astype(jnp.int32)
  off2 = (bstart[None, :] + piece).astype(jnp.int32)
  tstart = (bstart + used).astype(jnp.int32)
  tlen = (blen - used).astype(jnp.int32)
  bchunks = (blen // 128).astype(jnp.int32)
  bsrc, bloc, bw = _sc_bucket(src3, dst3, w3, off2, tstart, tlen)

  deg_parts = _sc_deg(dst3, w3)
  deg = deg_parts[0, :N_NODES] + deg_parts[1, :N_NODES] + 1.0
  dinv = jnp.where(deg > 0, lax.rsqrt(deg), 0.0)
  dv = dinv.reshape(N_NODES, 1)

  y1 = _tc_matmul(x, W1, dv)
  p1 = _sc_spmm(y1, bsrc, bloc, bw, bstart, bchunks).reshape(N_PAD, D)
  y2 = _tc_mid(p1, y1, dv, b1.reshape(1, D), W2)
  p2 = _sc_spmm(y2, bsrc, bloc, bw, bstart, bchunks).reshape(N_PAD, D)

  out = _tc_final(p2, y2, dv, b2.reshape(1, D), bid, speed,
                  Ws, bs.reshape(1, 4), Wl1[:D], Wl1[D:],
                  bl1.reshape(1, N_GRAPHS), Wl2, bl2.reshape(1, N_GRAPHS))
  return out
